# Initial kernel scaffold; baseline (speedup 1.0000x reference)
import jax
import jax.numpy as jnp
from jax.experimental import pallas as pl


def kernel(x, table):
    raise NotImplementedError("write your pallas kernel here")


# speedup vs baseline: 1.1037x; 1.1037x over previous
import functools

import jax
import jax.numpy as jnp
from jax import lax
from jax.experimental import pallas as pl
from jax.experimental.pallas import tpu as pltpu
from jax.experimental.pallas import tpu_sc as plsc

_BATCH = 16384
_HIST = 50
_D = 32
_B = _BATCH * _HIST

_info = plsc.get_sparse_core_info()
_NC, _NS = _info.num_cores, _info.num_subcores
_NW = _NC * _NS
_BPW = _B // _NW
_CH = 1600
_NCHUNK = _BPW // _CH


def _make_gather():
  mesh = plsc.VectorSubcoreMesh(core_axis_name="c", subcore_axis_name="s")

  @functools.partial(
      pl.kernel,
      mesh=mesh,
      out_type=jax.ShapeDtypeStruct((_B, _D), jnp.float32),
      scratch_types=[
          pltpu.VMEM((_CH,), jnp.int32),
          pltpu.VMEM((_CH, _D), jnp.float32),
          pltpu.SemaphoreType.DMA,
      ],
      compiler_params=pltpu.CompilerParams(use_tc_tiling_on_sc=False),
  )
  def gather_kernel(table_hbm, idx_hbm, out_hbm, idx_v, rows_v, sem):
    wid = lax.axis_index("s") * _NC + lax.axis_index("c")
    base = wid * _BPW

    def body(c, carry):
      off = base + c * _CH
      pltpu.sync_copy(idx_hbm.at[pl.ds(off, _CH)], idx_v)
      pltpu.async_copy(table_hbm.at[idx_v], rows_v, sem).wait()
      pltpu.sync_copy(rows_v, out_hbm.at[pl.ds(off, _CH)])
      return carry

    lax.fori_loop(0, _NCHUNK, body, 0, unroll=False)

  return gather_kernel


_gather = _make_gather()


def kernel(x, table):
  idx = x.reshape(_B)
  out = _gather(table, idx)
  return out.reshape(_BATCH, _HIST, _D)


# speedup vs baseline: 1.1098x; 1.0055x over previous
import functools

import jax
import jax.numpy as jnp
from jax import lax
from jax.experimental import pallas as pl
from jax.experimental.pallas import tpu as pltpu
from jax.experimental.pallas import tpu_sc as plsc

_BATCH = 16384
_HIST = 50
_D = 32
_B = _BATCH * _HIST

_info = plsc.get_sparse_core_info()
_NC, _NS = _info.num_cores, _info.num_subcores
_NW = _NC * _NS
_BPW = _B // _NW
_CH = 1600
_NCHUNK = _BPW // _CH
_NBUF = 2


def _make_gather():
  mesh = plsc.VectorSubcoreMesh(core_axis_name="c", subcore_axis_name="s")

  @functools.partial(
      pl.kernel,
      mesh=mesh,
      out_type=jax.ShapeDtypeStruct((_B, _D), jnp.float32),
      scratch_types=[
          pltpu.VMEM((_BPW,), jnp.int32),
          pltpu.VMEM((_NBUF, _CH, _D), jnp.float32),
          pltpu.SemaphoreType.DMA,
          pltpu.SemaphoreType.DMA,
          pltpu.SemaphoreType.DMA,
          pltpu.SemaphoreType.DMA,
      ],
      compiler_params=pltpu.CompilerParams(use_tc_tiling_on_sc=False),
  )
  def gather_kernel(table_hbm, idx_hbm, out_hbm, idx_v, rows_v, gs0, gs1,
                    ss0, ss1):
    gsems = (gs0, gs1)
    ssems = (ss0, ss1)
    wid = lax.axis_index("s") * _NC + lax.axis_index("c")
    base = wid * _BPW

    def gather_copy(c, b):
      return pltpu.make_async_copy(
          table_hbm.at[idx_v.at[pl.ds(c * _CH, _CH)]], rows_v.at[b], gsems[b])

    def store_copy(c, b):
      return pltpu.make_async_copy(
          rows_v.at[b], out_hbm.at[pl.ds(base + c * _CH, _CH)], ssems[b])

    pltpu.sync_copy(idx_hbm.at[pl.ds(base, _BPW)], idx_v)
    gather_copy(0, 0).start()

    def outer(g, carry):
      for b in range(_NBUF):
        c = g * _NBUF + b
        gather_copy(c, b).wait()
        store_copy(c, b).start()

        @pl.when(c >= 1)
        def _():
          store_copy(c - 1, 1 - b).wait()

        @pl.when(c < _NCHUNK - 1)
        def _():
          gather_copy(c + 1, 1 - b).start()

      return carry

    lax.fori_loop(0, _NCHUNK // _NBUF, outer, 0, unroll=False)
    store_copy(_NCHUNK - 1, (_NCHUNK - 1) % _NBUF).wait()

  return gather_kernel


_gather = _make_gather()


def kernel(x, table):
  idx = x.reshape(_B)
  out = _gather(table, idx)
  return out.reshape(_BATCH, _HIST, _D)


# speedup vs baseline: 1.4982x; 1.3500x over previous
import functools

import jax
import jax.numpy as jnp
from jax import lax
from jax.experimental import pallas as pl
from jax.experimental.pallas import tpu as pltpu
from jax.experimental.pallas import tpu_sc as plsc

_BATCH = 16384
_HIST = 50
_D = 32
_B = _BATCH * _HIST

_info = plsc.get_sparse_core_info()
_NC, _NS = _info.num_cores, _info.num_subcores
_NW = _NC * _NS
_BATCH_PW = _BATCH // _NW
_BPW = _B // _NW
_CB = 16
_CH = _CB * _HIST
_NCHUNK = _BATCH_PW // _CB
_NBUF = 2


def _make_gather():
  mesh = plsc.VectorSubcoreMesh(core_axis_name="c", subcore_axis_name="s")

  @functools.partial(
      pl.kernel,
      mesh=mesh,
      out_type=jax.ShapeDtypeStruct((_HIST * _D, _BATCH), jnp.float32),
      scratch_types=[
          pltpu.VMEM((_BPW,), jnp.int32),
          pltpu.VMEM((_NBUF, _CH, _D), jnp.float32),
          pltpu.VMEM((_NBUF, _HIST * _D, _CB), jnp.float32),
          pltpu.SemaphoreType.DMA,
          pltpu.SemaphoreType.DMA,
          pltpu.SemaphoreType.DMA,
          pltpu.SemaphoreType.DMA,
      ],
      compiler_params=pltpu.CompilerParams(
          use_tc_tiling_on_sc=False, needs_layout_passes=False),
  )
  def gather_kernel(table_hbm, idx_hbm, out_hbm, idx_v, rows_v, tout_v,
                    gs0, gs1, ss0, ss1):
    gsems = (gs0, gs1)
    ssems = (ss0, ss1)
    wid = lax.axis_index("s") * _NC + lax.axis_index("c")
    base = wid * _BPW
    bbase = wid * _BATCH_PW

    def gather_copy(c, b):
      return pltpu.make_async_copy(
          table_hbm.at[idx_v.at[pl.ds(c * _CH, _CH)]], rows_v.at[b], gsems[b])

    def store_copy(c, b):
      return pltpu.make_async_copy(
          tout_v.at[b], out_hbm.at[:, pl.ds(bbase + c * _CB, _CB)], ssems[b])

    pltpu.sync_copy(idx_hbm.at[pl.ds(base, _BPW)], idx_v)
    gather_copy(0, 0).start()

    lane = lax.iota(jnp.int32, 16)
    zeros = jnp.zeros((16,), jnp.int32)

    def outer(g, carry):
      for b in range(_NBUF):
        c = g * _NBUF + b
        gather_copy(c, b).wait()

        @pl.when(c < _NCHUNK - 1)
        def _():
          gather_copy(c + 1, 1 - b).start()

        @pl.when(c >= _NBUF)
        def _():
          store_copy(c - _NBUF, b).wait()

        rows = rows_v.at[b]
        tout = tout_v.at[b]

        def transpose_t(t, carry2):
          src_rows = lane * _HIST + t
          tbase = t * _D
          for d in range(_D):
            vals = plsc.load_gather(rows, [src_rows, zeros + d])
            plsc.store_scatter(tout, [zeros + (tbase + d), lane], vals)
          return carry2

        lax.fori_loop(0, _HIST, transpose_t, 0, unroll=False)
        store_copy(c, b).start()

      return carry

    lax.fori_loop(0, _NCHUNK // _NBUF, outer, 0, unroll=False)
    store_copy(_NCHUNK - 2, 0).wait()
    store_copy(_NCHUNK - 1, 1).wait()

  return gather_kernel


_gather = _make_gather()


def kernel(x, table):
  idx = x.reshape(_B)
  out2d = _gather(table, idx)
  out = out2d.reshape(_HIST, _D, _BATCH).transpose(2, 0, 1)
  return out


# speedup vs baseline: 2.6602x; 1.7756x over previous
import functools

import jax
import jax.numpy as jnp
from jax import lax
from jax.experimental import pallas as pl
from jax.experimental.pallas import tpu as pltpu
from jax.experimental.pallas import tpu_sc as plsc

_BATCH = 16384
_HIST = 50
_D = 32
_B = _BATCH * _HIST

_info = plsc.get_sparse_core_info()
_NC, _NS = _info.num_cores, _info.num_subcores
_NW = _NC * _NS
_BATCH_PW = _BATCH // _NW
_BPW = _B // _NW
_CB = _BATCH_PW
_NV = _CB // 16
_NBUF = 2


def _make_gather():
  mesh = plsc.VectorSubcoreMesh(core_axis_name="c", subcore_axis_name="s")

  @functools.partial(
      pl.kernel,
      mesh=mesh,
      out_type=jax.ShapeDtypeStruct((_HIST * _D, _BATCH), jnp.float32),
      scratch_types=[
          pltpu.VMEM((_BPW,), jnp.int32),
          pltpu.VMEM((_BPW,), jnp.int32),
          pltpu.VMEM((_NBUF, _CB, _D), jnp.float32),
          pltpu.VMEM((_NBUF, _D, _CB + 1), jnp.float32),
          pltpu.SemaphoreType.DMA,
          pltpu.SemaphoreType.DMA,
          pltpu.SemaphoreType.DMA,
          pltpu.SemaphoreType.DMA,
      ],
      compiler_params=pltpu.CompilerParams(
          use_tc_tiling_on_sc=False, needs_layout_passes=False),
  )
  def gather_kernel(table_hbm, idx_hbm, out_hbm, idx_v, cidx_v, rows_v,
                    tout_v, gs0, gs1, ss0, ss1):
    gsems = (gs0, gs1)
    ssems = (ss0, ss1)
    wid = lax.axis_index("s") * _NC + lax.axis_index("c")
    base = wid * _BPW
    bbase = wid * _BATCH_PW

    lane = lax.iota(jnp.int32, 16)
    lane16 = lane + 16
    lane_h = lane * _HIST

    def gather_copy(t, b):
      return pltpu.make_async_copy(
          table_hbm.at[cidx_v.at[pl.ds(t * _CB, _CB)]],
          rows_v.at[b], gsems[b])

    def store_copy(t, b):
      return pltpu.make_async_copy(
          tout_v.at[b].at[:, pl.ds(0, _CB)],
          out_hbm.at[pl.ds(t * _D, _D), pl.ds(bbase, _CB)], ssems[b])

    pltpu.sync_copy(idx_hbm.at[pl.ds(base, _BPW)], idx_v)

    @plsc.parallel_loop(0, _HIST * (_NV // 8))
    def _(k):
      t = k >> 2
      vg = k & 3
      vt = lane_h + t
      vals = []
      for j in range(8):
        v = vg * 8 + j
        vals.append(
            plsc.load_gather(idx_v.at[pl.ds(v * 16 * _HIST, 16 * _HIST)],
                             [vt]))
      for j in range(8):
        cidx_v[pl.ds(t * _CB + (vg * 8 + j) * 16, 16)] = vals[j]

    gather_copy(0, 0).start()
    gather_copy(1, 1).start()

    def body(t, b):
      gather_copy(t, b).wait()

      @pl.when(t >= _NBUF)
      def _():
        store_copy(t - _NBUF, b).wait()

      rows = rows_v.at[b]
      tout = tout_v.at[b]

      @plsc.parallel_loop(0, _CB, unroll=4)
      def _(j):
        v0 = rows[j, pl.ds(0, 16)]
        v1 = rows[j, pl.ds(16, 16)]
        cj = jnp.zeros((16,), jnp.int32) + j
        plsc.store_scatter(tout, [lane, cj], v0)
        plsc.store_scatter(tout, [lane16, cj], v1)

      store_copy(t, b).start()

      @pl.when(t + _NBUF < _HIST)
      def _():
        gather_copy(t + _NBUF, b).start()

    def outer(g, carry):
      for b in range(_NBUF):
        body(g * _NBUF + b, b)
      return carry

    lax.fori_loop(0, _HIST // _NBUF, outer, 0, unroll=False)
    store_copy(_HIST - 2, 0).wait()
    store_copy(_HIST - 1, 1).wait()

  return gather_kernel


_gather = _make_gather()


def kernel(x, table):
  idx = x.reshape(_B)
  out2d = _gather(table, idx)
  out = out2d.reshape(_HIST, _D, _BATCH).transpose(2, 0, 1)
  return out
